# baseline (speedup 1.0000x reference)
import functools

import jax
import jax.numpy as jnp
from jax import lax
from jax.experimental import pallas as pl
from jax.experimental.pallas import tpu as pltpu
from jax.experimental.pallas import tpu_sc as plsc

PROB_DO = 0.5
CHUNK_ROWS = 16


def kernel(x):
    batch, width = x.shape
    info = plsc.get_sparse_core_info()
    nc, ns, lanes = info.num_cores, info.num_subcores, info.num_lanes
    nw = nc * ns
    rows_per_w = batch // nw
    n_chunks = rows_per_w // CHUNK_ROWS
    vecs_per_row = width // lanes

    with jax.ensure_compile_time_eval():
        mask = jax.random.uniform(jax.random.key(1), (batch,)) < PROB_DO
        c = jnp.broadcast_to(
            (1.0 - mask.astype(x.dtype))[:, None], (batch, lanes)
        )

    mesh = plsc.VectorSubcoreMesh(core_axis_name="c", subcore_axis_name="s")

    @functools.partial(
        pl.kernel,
        out_type=jax.ShapeDtypeStruct((batch, width), x.dtype),
        mesh=mesh,
        scratch_types=[
            pltpu.VMEM((rows_per_w, lanes), x.dtype),
            pltpu.VMEM((2, CHUNK_ROWS, width), x.dtype),
            pltpu.VMEM_SHARED((ns, 2, CHUNK_ROWS, width), x.dtype),
            pltpu.SemaphoreType.DMA,
            pltpu.SemaphoreType.DMA,
            pltpu.SemaphoreType.DMA,
            pltpu.SemaphoreType.DMA,
            pltpu.SemaphoreType.DMA,
            pltpu.SemaphoreType.DMA,
        ],
    )
    def _sc(c_hbm, x_hbm, o_hbm, c_v, ibuf, shared, si0, si1, sx0, sx1, so0, so1):
        s = lax.axis_index("s")
        wid = s * nc + lax.axis_index("c")
        base = wid * rows_per_w
        sins, sxbars, souts = (si0, si1), (sx0, sx1), (so0, so1)

        pltpu.sync_copy(c_hbm.at[pl.ds(base, rows_per_w)], c_v)

        def islice(g):
            return x_hbm.at[pl.ds(base + g * CHUNK_ROWS, CHUNK_ROWS)]

        def oslice(g):
            return o_hbm.at[pl.ds(base + g * CHUNK_ROWS, CHUNK_ROWS)]

        pltpu.async_copy(islice(0), ibuf.at[0], si0)

        def outer(gg, _):
            for b in range(2):
                g = gg * 2 + b
                ib = ibuf.at[b]
                sb = shared.at[s, b]

                pltpu.make_async_copy(islice(g), ib, sins[b]).wait()

                @pl.when(g >= 2)
                def _drain_out():
                    pltpu.make_async_copy(sb, oslice(g - 2), souts[b]).wait()

                def row(r, _):
                    cv = c_v[g * CHUNK_ROWS + r, :]
                    for j in range(vecs_per_row):
                        v = ib[r, pl.ds(j * lanes, lanes)]
                        ib[r, pl.ds(j * lanes, lanes)] = jnp.maximum(v, v * cv)
                    return 0

                lax.fori_loop(0, CHUNK_ROWS, row, 0)

                pltpu.async_copy(ib, sb, sxbars[b])

                @pl.when(g >= 1)
                def _out_prev():
                    pltpu.make_async_copy(ibuf.at[1 - b], shared.at[s, 1 - b],
                                          sxbars[1 - b]).wait()
                    pltpu.async_copy(shared.at[s, 1 - b], oslice(g - 1),
                                     souts[1 - b])

                @pl.when(g + 1 < n_chunks)
                def _next_in():
                    pltpu.async_copy(islice(g + 1), ibuf.at[1 - b], sins[1 - b])

            return 0

        lax.fori_loop(0, n_chunks // 2, outer, 0)

        lb = (n_chunks - 1) % 2
        pltpu.make_async_copy(ibuf.at[lb], shared.at[s, lb], sxbars[lb]).wait()
        pltpu.async_copy(shared.at[s, lb], oslice(n_chunks - 1), souts[lb])
        pltpu.make_async_copy(shared.at[s, 1 - lb], oslice(n_chunks - 2),
                              souts[1 - lb]).wait()
        pltpu.make_async_copy(shared.at[s, lb], oslice(n_chunks - 1),
                              souts[lb]).wait()

    return _sc(c, x)
